# baseline (speedup 1.0000x reference)
import functools

import numpy as np
import jax
import jax.numpy as jnp
from jax import lax
from jax.experimental import pallas as pl
from jax.experimental.pallas import tpu as pltpu
from jax.experimental.pallas import tpu_sc as plsc

EMBED = 128
WINDOW = 2048
BATCH = 4
TOTAL = BATCH * WINDOW
SCALE = float(np.sqrt(np.float32(EMBED)))

NC = 2
NS = 16
NW = NC * NS
BPW = TOTAL // NW
LANES = 16
CHUNKS = (16, 16, 32, 32, 32, 32, 48, 48)
NCHUNK = len(CHUNKS)
OFFS = (0, 16, 32, 64, 96, 128, 160, 208)
PWORDS = EMBED


def _pos_encoding_packed() -> np.ndarray:
    half = EMBED // 2
    positions = np.arange(WINDOW, dtype=np.float32)[:, None]
    depths = np.arange(half, dtype=np.float32)[None, :] / np.float32(half)
    angle_rates = 1.0 / (10000.0 ** depths)
    angle_rads = positions * angle_rates
    pos = np.concatenate([np.sin(angle_rads), np.cos(angle_rads)], axis=-1)
    pos = pos.astype(np.float32) / np.float32(SCALE)
    return pos.reshape(WINDOW, EMBED)


_POS_PACKED = _pos_encoding_packed()

_mesh = plsc.VectorSubcoreMesh(core_axis_name="c", subcore_axis_name="s")


@functools.partial(
    pl.kernel,
    mesh=_mesh,
    compiler_params=pltpu.CompilerParams(use_tc_tiling_on_sc=True),
    out_type=jax.ShapeDtypeStruct((TOTAL, EMBED), jnp.float32),
    scratch_types=[
        pltpu.VMEM((BPW,), jnp.int32),
        pltpu.VMEM((BPW, EMBED), jnp.float32),
        pltpu.SemaphoreType.DMA,
    ]
    + [pltpu.SemaphoreType.DMA] * NCHUNK
    + [pltpu.SemaphoreType.DMA] * NCHUNK
    + [pltpu.SemaphoreType.DMA] * NCHUNK,
)
def _emb_kernel(x_hbm, table_hbm, pos_hbm, out_hbm, idx_v, rows_v,
                sem_p, *sems):
    gsems = sems[:NCHUNK]
    wsems = sems[NCHUNK:2 * NCHUNK]
    psems = sems[2 * NCHUNK:]
    wid = lax.axis_index("s") * NC + lax.axis_index("c")
    base = wid * BPW
    b = base // WINDOW
    l0 = lax.rem(base, WINDOW)
    pcps = [
        pltpu.async_copy(
            pos_hbm.at[pl.ds(l0 + OFFS[c], CHUNKS[c])],
            rows_v.at[pl.ds(OFFS[c], CHUNKS[c])],
            psems[c])
        for c in range(NCHUNK)
    ]
    pltpu.sync_copy(x_hbm.at[b, pl.ds(l0, BPW)], idx_v)
    gcps = []
    for c in range(NCHUNK):
        pcps[c].wait()
        gcps.append(pltpu.async_copy(
            table_hbm.at[idx_v.at[pl.ds(OFFS[c], CHUNKS[c])]],
            rows_v.at[pl.ds(OFFS[c], CHUNKS[c])],
            gsems[c],
            add=True))

    wcps = []
    for c in range(NCHUNK):
        gcps[c].wait()

        def row_step(j, carry, _c=c):
            r = OFFS[_c] + j
            for k in range(EMBED // LANES):
                sl = pl.ds(k * LANES, LANES)
                rows_v[r, sl] = rows_v[r, sl] * SCALE
            return carry

        lax.fori_loop(0, CHUNKS[c], row_step, 0)
        wcps.append(pltpu.async_copy(
            rows_v.at[pl.ds(OFFS[c], CHUNKS[c])],
            out_hbm.at[pl.ds(base + OFFS[c], CHUNKS[c])],
            wsems[c]))
    for w in wcps:
        w.wait()


def kernel(x, table):
    xi = x.astype(jnp.int32)
    zero = (xi[0, 0] >> 31).astype(jnp.float32)
    pos = jnp.asarray(_POS_PACKED) + zero
    out = _emb_kernel(xi, table, pos)
    return out.reshape(BATCH, WINDOW, EMBED)


# speedup vs baseline: 1.0304x; 1.0304x over previous
import functools

import numpy as np
import jax
import jax.numpy as jnp
from jax import lax
from jax.experimental import pallas as pl
from jax.experimental.pallas import tpu as pltpu
from jax.experimental.pallas import tpu_sc as plsc

EMBED = 128
WINDOW = 2048
BATCH = 4
TOTAL = BATCH * WINDOW
SCALE = float(np.sqrt(np.float32(EMBED)))

NC = 2
NS = 16
NW = NC * NS
BPW = TOTAL // NW
LANES = 16
CHUNKS = (16, 16, 32, 32, 32, 32, 48, 48)
NCHUNK = len(CHUNKS)
OFFS = (0, 16, 32, 64, 96, 128, 160, 208)
PWORDS = EMBED


def _pos_encoding_packed() -> np.ndarray:
    half = EMBED // 2
    positions = np.arange(WINDOW, dtype=np.float32)[:, None]
    depths = np.arange(half, dtype=np.float32)[None, :] / np.float32(half)
    angle_rates = 1.0 / (10000.0 ** depths)
    angle_rads = positions * angle_rates
    pos = np.concatenate([np.sin(angle_rads), np.cos(angle_rads)], axis=-1)
    pos = pos.astype(np.float32) / np.float32(SCALE)
    return pos.reshape(WINDOW, EMBED)


_POS_PACKED = _pos_encoding_packed()

_mesh = plsc.VectorSubcoreMesh(core_axis_name="c", subcore_axis_name="s")


@functools.partial(
    pl.kernel,
    mesh=_mesh,
    out_type=jax.ShapeDtypeStruct((TOTAL, EMBED), jnp.float32),
    scratch_types=[
        pltpu.VMEM((BPW,), jnp.int32),
        pltpu.VMEM((BPW, EMBED), jnp.float32),
        pltpu.SemaphoreType.DMA,
    ]
    + [pltpu.SemaphoreType.DMA] * NCHUNK
    + [pltpu.SemaphoreType.DMA] * NCHUNK
    + [pltpu.SemaphoreType.DMA] * NCHUNK,
)
def _emb_kernel(x_hbm, table_hbm, pos_hbm, out_hbm, idx_v, rows_v,
                sem_p, *sems):
    gsems = sems[:NCHUNK]
    wsems = sems[NCHUNK:2 * NCHUNK]
    psems = sems[2 * NCHUNK:]
    wid = lax.axis_index("s") * NC + lax.axis_index("c")
    base = wid * BPW
    b = base // WINDOW
    l0 = lax.rem(base, WINDOW)
    pcps = [
        pltpu.async_copy(
            pos_hbm.at[pl.ds(l0 + OFFS[c], CHUNKS[c])],
            rows_v.at[pl.ds(OFFS[c], CHUNKS[c])],
            psems[c])
        for c in range(NCHUNK)
    ]
    pltpu.sync_copy(x_hbm.at[b, pl.ds(l0, BPW)], idx_v)
    gcps = []
    for c in range(NCHUNK):
        pcps[c].wait()
        gcps.append(pltpu.async_copy(
            table_hbm.at[idx_v.at[pl.ds(OFFS[c], CHUNKS[c])]],
            rows_v.at[pl.ds(OFFS[c], CHUNKS[c])],
            gsems[c],
            add=True))

    wcps = []
    for c in range(NCHUNK):
        gcps[c].wait()

        def row_step(j, carry, _c=c):
            r = OFFS[_c] + j
            for k in range(EMBED // LANES):
                sl = pl.ds(k * LANES, LANES)
                rows_v[r, sl] = rows_v[r, sl] * SCALE
            return carry

        lax.fori_loop(0, CHUNKS[c], row_step, 0)
        wcps.append(pltpu.async_copy(
            rows_v.at[pl.ds(OFFS[c], CHUNKS[c])],
            out_hbm.at[pl.ds(base + OFFS[c], CHUNKS[c])],
            wsems[c]))
    for w in wcps:
        w.wait()


def kernel(x, table):
    pos = jnp.asarray(_POS_PACKED)
    out = _emb_kernel(x.astype(jnp.int32), table, pos)
    return out.reshape(BATCH, WINDOW, EMBED)


# speedup vs baseline: 1.0349x; 1.0043x over previous
import functools

import numpy as np
import jax
import jax.numpy as jnp
from jax import lax
from jax.experimental import pallas as pl
from jax.experimental.pallas import tpu as pltpu
from jax.experimental.pallas import tpu_sc as plsc

EMBED = 128
WINDOW = 2048
BATCH = 4
TOTAL = BATCH * WINDOW
SCALE = float(np.sqrt(np.float32(EMBED)))

NC = 2
NS = 16
NW = NC * NS
BPW = TOTAL // NW
LANES = 16
CHUNKS = (16, 16, 16, 32, 32, 48, 48, 48)
NCHUNK = len(CHUNKS)
OFFS = (0, 16, 32, 48, 80, 112, 160, 208)
PWORDS = EMBED


def _pos_encoding_packed() -> np.ndarray:
    half = EMBED // 2
    positions = np.arange(WINDOW, dtype=np.float32)[:, None]
    depths = np.arange(half, dtype=np.float32)[None, :] / np.float32(half)
    angle_rates = 1.0 / (10000.0 ** depths)
    angle_rads = positions * angle_rates
    pos = np.concatenate([np.sin(angle_rads), np.cos(angle_rads)], axis=-1)
    pos = pos.astype(np.float32) / np.float32(SCALE)
    return pos.reshape(WINDOW, EMBED)


_POS_PACKED = _pos_encoding_packed()

_mesh = plsc.VectorSubcoreMesh(core_axis_name="c", subcore_axis_name="s")


@functools.partial(
    pl.kernel,
    mesh=_mesh,
    out_type=jax.ShapeDtypeStruct((TOTAL, EMBED), jnp.float32),
    scratch_types=[
        pltpu.VMEM((BPW,), jnp.int32),
        pltpu.VMEM((BPW, EMBED), jnp.float32),
        pltpu.SemaphoreType.DMA,
    ]
    + [pltpu.SemaphoreType.DMA] * NCHUNK
    + [pltpu.SemaphoreType.DMA] * NCHUNK
    + [pltpu.SemaphoreType.DMA] * NCHUNK,
)
def _emb_kernel(x_hbm, table_hbm, pos_hbm, out_hbm, idx_v, rows_v,
                sem_p, *sems):
    gsems = sems[:NCHUNK]
    wsems = sems[NCHUNK:2 * NCHUNK]
    psems = sems[2 * NCHUNK:]
    wid = lax.axis_index("s") * NC + lax.axis_index("c")
    base = wid * BPW
    b = base // WINDOW
    l0 = lax.rem(base, WINDOW)
    pcps = [
        pltpu.async_copy(
            pos_hbm.at[pl.ds(l0 + OFFS[c], CHUNKS[c])],
            rows_v.at[pl.ds(OFFS[c], CHUNKS[c])],
            psems[c])
        for c in range(NCHUNK)
    ]
    pltpu.sync_copy(x_hbm.at[b, pl.ds(l0, BPW)], idx_v)
    gcps = []
    for c in range(NCHUNK):
        pcps[c].wait()
        gcps.append(pltpu.async_copy(
            table_hbm.at[idx_v.at[pl.ds(OFFS[c], CHUNKS[c])]],
            rows_v.at[pl.ds(OFFS[c], CHUNKS[c])],
            gsems[c],
            add=True))

    wcps = []
    for c in range(NCHUNK):
        gcps[c].wait()

        def row_step(j, carry, _c=c):
            r = OFFS[_c] + j
            for k in range(EMBED // LANES):
                sl = pl.ds(k * LANES, LANES)
                rows_v[r, sl] = rows_v[r, sl] * SCALE
            return carry

        lax.fori_loop(0, CHUNKS[c], row_step, 0)
        wcps.append(pltpu.async_copy(
            rows_v.at[pl.ds(OFFS[c], CHUNKS[c])],
            out_hbm.at[pl.ds(base + OFFS[c], CHUNKS[c])],
            wsems[c]))
    for w in wcps:
        w.wait()


def kernel(x, table):
    pos = jnp.asarray(_POS_PACKED)
    out = _emb_kernel(x.astype(jnp.int32), table, pos)
    return out.reshape(BATCH, WINDOW, EMBED)


# speedup vs baseline: 1.0399x; 1.0049x over previous
import functools

import numpy as np
import jax
import jax.numpy as jnp
from jax import lax
from jax.experimental import pallas as pl
from jax.experimental.pallas import tpu as pltpu
from jax.experimental.pallas import tpu_sc as plsc

EMBED = 128
WINDOW = 2048
BATCH = 4
TOTAL = BATCH * WINDOW
SCALE = float(np.sqrt(np.float32(EMBED)))

NC = 2
NS = 16
NW = NC * NS
BPW = TOTAL // NW
LANES = 16
CHUNKS = (16, 16, 16, 32, 32, 48, 48, 48)
NCHUNK = len(CHUNKS)
OFFS = (0, 16, 32, 48, 80, 112, 160, 208)


def _pos_over_scale() -> np.ndarray:
    half = EMBED // 2
    positions = np.arange(WINDOW, dtype=np.float32)[:, None]
    depths = np.arange(half, dtype=np.float32)[None, :] / np.float32(half)
    angle_rates = 1.0 / (10000.0 ** depths)
    angle_rads = positions * angle_rates
    pos = np.concatenate([np.sin(angle_rads), np.cos(angle_rads)], axis=-1)
    return (pos.astype(np.float32) / np.float32(SCALE)).reshape(WINDOW, EMBED)


_POS = _pos_over_scale()

_mesh = plsc.VectorSubcoreMesh(core_axis_name="c", subcore_axis_name="s")


@functools.partial(
    pl.kernel,
    mesh=_mesh,
    out_type=jax.ShapeDtypeStruct((TOTAL, EMBED), jnp.float32),
    scratch_types=[
        pltpu.VMEM((BPW,), jnp.int32),
        pltpu.VMEM((BPW, EMBED), jnp.float32),
    ]
    + [pltpu.SemaphoreType.DMA] * NCHUNK
    + [pltpu.SemaphoreType.DMA] * NCHUNK
    + [pltpu.SemaphoreType.DMA] * NCHUNK,
)
def _emb_kernel(x_hbm, table_hbm, pos_hbm, out_hbm, idx_v, rows_v, *sems):
    gsems = sems[:NCHUNK]
    wsems = sems[NCHUNK:2 * NCHUNK]
    psems = sems[2 * NCHUNK:]
    wid = lax.axis_index("s") * NC + lax.axis_index("c")
    base = wid * BPW
    b = base // WINDOW
    l0 = lax.rem(base, WINDOW)
    pcps = [
        pltpu.async_copy(
            pos_hbm.at[pl.ds(l0 + OFFS[c], CHUNKS[c])],
            rows_v.at[pl.ds(OFFS[c], CHUNKS[c])],
            psems[c])
        for c in range(NCHUNK)
    ]
    pltpu.sync_copy(x_hbm.at[b, pl.ds(l0, BPW)], idx_v)
    gcps = []
    for c in range(NCHUNK):
        pcps[c].wait()
        gcps.append(pltpu.async_copy(
            table_hbm.at[idx_v.at[pl.ds(OFFS[c], CHUNKS[c])]],
            rows_v.at[pl.ds(OFFS[c], CHUNKS[c])],
            gsems[c],
            add=True))

    wcps = []
    for c in range(NCHUNK):
        gcps[c].wait()

        def row_step(j, carry, _c=c):
            r = OFFS[_c] + j
            for k in range(EMBED // LANES):
                sl = pl.ds(k * LANES, LANES)
                rows_v[r, sl] = rows_v[r, sl] * SCALE
            return carry

        lax.fori_loop(0, CHUNKS[c], row_step, 0)
        wcps.append(pltpu.async_copy(
            rows_v.at[pl.ds(OFFS[c], CHUNKS[c])],
            out_hbm.at[pl.ds(base + OFFS[c], CHUNKS[c])],
            wsems[c]))
    for w in wcps:
        w.wait()


def kernel(x, table):
    pos = jnp.asarray(_POS)
    out = _emb_kernel(x.astype(jnp.int32), table, pos)
    return out.reshape(BATCH, WINDOW, EMBED)
